# baseline (speedup 1.0000x reference)
import functools

import jax
import jax.numpy as jnp
from jax.experimental import pallas as pl
from jax.experimental.pallas import tpu as pltpu


_LANES = 128
_NC = 16


def _fused_gnn_kernel(W_ref, x_ref, mask_ref,
                      zt0_ref, b0_ref, zt1_ref, b1_ref, zt2_ref, b2_ref,
                      ztl_ref, bl_ref, out_ref, wbf_ref, z0_ref, u0_ref,
                      nf: int, d_out: int):
    b_id = pl.program_id(0)
    c = pl.program_id(1)
    n = wbf_ref.shape[0]
    nj = wbf_ref.shape[1]
    rc = n // _NC

    def make_z(cur, zt):
        zwide = jnp.dot(cur, zt, preferred_element_type=jnp.float32)
        return zwide.reshape(nj, _LANES).astype(jnp.bfloat16)

    def act(u, b_row, mask):
        v = u + b_row
        lane = jax.lax.broadcasted_iota(jnp.int32, v.shape, 1)
        v = jnp.where(lane < nf, jnp.maximum(v, 0.0), v)
        return v * mask

    def big_dot_to_u0(z):
        for rb in range(_NC):
            u0_ref[pl.ds(rb * rc, rc)] = jnp.dot(
                wbf_ref[pl.ds(rb * rc, rc)], z,
                preferred_element_type=jnp.float32)

    @pl.when(c == 0)
    def _():
        z0_ref[...] = make_z(x_ref[b_id], zt0_ref[...])

    wblk = W_ref[0].astype(jnp.bfloat16)
    wbf_ref[pl.ds(c * rc, rc)] = wblk
    u0_ref[pl.ds(c * rc, rc)] = jnp.dot(wblk, z0_ref[...],
                                        preferred_element_type=jnp.float32)

    @pl.when(c == _NC - 1)
    def _():
        mask = mask_ref[b_id]
        cur = act(u0_ref[...], b0_ref[...], mask)
        for zt_ref, br_ref in ((zt1_ref, b1_ref), (zt2_ref, b2_ref)):
            z = make_z(cur, zt_ref[...])
            big_dot_to_u0(z)
            cur = act(u0_ref[...], br_ref[...], mask)
        zl = make_z(cur, ztl_ref[...])
        big_dot_to_u0(zl)
        out_ref[0] = ((u0_ref[...] + bl_ref[...]) * mask)[:, :d_out]


def _zproj(w1, w2, j, pad_rows):
    w = jnp.concatenate([w1, w2], axis=0)
    return _zpad(w, j, pad_rows)


def _zpad(w, j, pad_rows):
    k = w.shape[0]
    d = w.shape[1] // j
    zt = w.reshape(k, j, d).transpose(2, 1, 0)
    zt = jnp.pad(zt, ((0, _LANES - d if pad_rows else 0),
                      (0, 0), (0, _LANES - k)))
    return zt.reshape(zt.shape[0], j * _LANES)


def kernel(W, x, mask, N_batch, fc1_w0, fc1_b0, fc2_w0, fc2_b0,
           fc1_w1, fc1_b1, fc2_w1, fc2_b1, fc1_w2, fc1_b2, fc2_w2, fc2_b2,
           fcl_w, fcl_b):
    bs, N, _, J = W.shape
    nf = fc1_b0.shape[0]
    d_out = fcl_w.shape[0]
    rc = N // _NC

    Wr = W.reshape(bs, N, N * J)

    zt0 = _zproj(fc1_w0, fc2_w0, J, pad_rows=False)
    zt1 = _zproj(fc1_w1, fc2_w1, J, pad_rows=True)
    zt2 = _zproj(fc1_w2, fc2_w2, J, pad_rows=True)

    def bpad(b1_, b2_):
        b = jnp.concatenate([b1_, b2_])
        return jnp.pad(b, (0, _LANES - b.shape[0])).reshape(1, _LANES)

    b0 = bpad(fc1_b0, fc2_b0)
    b1 = bpad(fc1_b1, fc2_b1)
    b2 = bpad(fc1_b2, fc2_b2)
    ztl = _zpad(fcl_w, J, pad_rows=True)
    bl = jnp.pad(fcl_b, (0, _LANES - d_out)).reshape(1, _LANES)

    def full(a):
        shape = a.shape
        return pl.BlockSpec(shape, lambda b, c: (0,) * len(shape))

    out = pl.pallas_call(
        functools.partial(_fused_gnn_kernel, nf=nf, d_out=d_out),
        grid=(bs, _NC),
        in_specs=[
            pl.BlockSpec((1, rc, N * J), lambda b, c: (b, c, 0)),
            pl.BlockSpec((bs, N, x.shape[-1]), lambda b, c: (0, 0, 0)),
            pl.BlockSpec((bs, N, 1), lambda b, c: (0, 0, 0)),
            full(zt0), full(b0), full(zt1), full(b1), full(zt2), full(b2),
            full(ztl), full(bl),
        ],
        out_specs=pl.BlockSpec((1, N, d_out), lambda b, c: (b, 0, 0)),
        out_shape=jax.ShapeDtypeStruct((bs, N, d_out), jnp.float32),
        scratch_shapes=[
            pltpu.VMEM((N, N * J), jnp.bfloat16),
            pltpu.VMEM((N * J, _LANES), jnp.bfloat16),
            pltpu.VMEM((N, _LANES), jnp.float32),
        ],
        compiler_params=pltpu.CompilerParams(
            dimension_semantics=("arbitrary", "arbitrary"),
            vmem_limit_bytes=128 * 1024 * 1024,
        ),
    )(Wr, x, mask, zt0, b0, zt1, b1, zt2, b2, ztl, bl)
    return out


# speedup vs baseline: 1.0319x; 1.0319x over previous
import functools

import jax
import jax.numpy as jnp
from jax.experimental import pallas as pl
from jax.experimental.pallas import tpu as pltpu


_LANES = 128
_NC = 16
_NQ = 4


def _fused_gnn_kernel(W0_ref, W1_ref, W2_ref, W3_ref, x_ref, mask_ref,
                      zt0_ref, b0_ref, zt1_ref, b1_ref, zt2_ref, b2_ref,
                      ztl_ref, bl_ref, out_ref, wbf_ref, z0_ref, u0_ref,
                      nf: int, d_out: int):
    b_id = pl.program_id(0)
    c = pl.program_id(1)
    n = wbf_ref.shape[0]
    nj = wbf_ref.shape[1]
    rc = n // _NC

    def make_z(cur, zt):
        zwide = jnp.dot(cur, zt, preferred_element_type=jnp.float32)
        return zwide.reshape(nj, _LANES).astype(jnp.bfloat16)

    def act(u, b_row, mask):
        v = u + b_row
        lane = jax.lax.broadcasted_iota(jnp.int32, v.shape, 1)
        v = jnp.where(lane < nf, jnp.maximum(v, 0.0), v)
        return v * mask

    def big_dot_to_u0(z):
        for rb in range(_NC):
            u0_ref[pl.ds(rb * rc, rc)] = jnp.dot(
                wbf_ref[pl.ds(rb * rc, rc)], z,
                preferred_element_type=jnp.float32)

    @pl.when(c == 0)
    def _():
        z0_ref[...] = make_z(x_ref[b_id], zt0_ref[...])

    for k, wq in enumerate((W0_ref, W1_ref, W2_ref, W3_ref)):
        wblk = wq[0].astype(jnp.bfloat16)
        row0 = (c * _NQ + k) * rc
        wbf_ref[pl.ds(row0, rc)] = wblk
        u0_ref[pl.ds(row0, rc)] = jnp.dot(wblk, z0_ref[...],
                                          preferred_element_type=jnp.float32)

    @pl.when(c == _NC // _NQ - 1)
    def _():
        mask = mask_ref[b_id]
        cur = act(u0_ref[...], b0_ref[...], mask)
        for zt_ref, br_ref in ((zt1_ref, b1_ref), (zt2_ref, b2_ref)):
            z = make_z(cur, zt_ref[...])
            big_dot_to_u0(z)
            cur = act(u0_ref[...], br_ref[...], mask)
        zl = make_z(cur, ztl_ref[...])
        big_dot_to_u0(zl)
        out_ref[0] = ((u0_ref[...] + bl_ref[...]) * mask)[:, :d_out]


def _zproj(w1, w2, j, pad_rows):
    w = jnp.concatenate([w1, w2], axis=0)
    return _zpad(w, j, pad_rows)


def _zpad(w, j, pad_rows):
    k = w.shape[0]
    d = w.shape[1] // j
    zt = w.reshape(k, j, d).transpose(2, 1, 0)
    zt = jnp.pad(zt, ((0, _LANES - d if pad_rows else 0),
                      (0, 0), (0, _LANES - k)))
    return zt.reshape(zt.shape[0], j * _LANES)


def kernel(W, x, mask, N_batch, fc1_w0, fc1_b0, fc2_w0, fc2_b0,
           fc1_w1, fc1_b1, fc2_w1, fc2_b1, fc1_w2, fc1_b2, fc2_w2, fc2_b2,
           fcl_w, fcl_b):
    bs, N, _, J = W.shape
    nf = fc1_b0.shape[0]
    d_out = fcl_w.shape[0]
    rc = N // _NC

    Wr = W.reshape(bs, N, N * J)

    zt0 = _zproj(fc1_w0, fc2_w0, J, pad_rows=False)
    zt1 = _zproj(fc1_w1, fc2_w1, J, pad_rows=True)
    zt2 = _zproj(fc1_w2, fc2_w2, J, pad_rows=True)

    def bpad(b1_, b2_):
        b = jnp.concatenate([b1_, b2_])
        return jnp.pad(b, (0, _LANES - b.shape[0])).reshape(1, _LANES)

    b0 = bpad(fc1_b0, fc2_b0)
    b1 = bpad(fc1_b1, fc2_b1)
    b2 = bpad(fc1_b2, fc2_b2)
    ztl = _zpad(fcl_w, J, pad_rows=True)
    bl = jnp.pad(fcl_b, (0, _LANES - d_out)).reshape(1, _LANES)

    def full(a):
        shape = a.shape
        return pl.BlockSpec(shape, lambda b, c: (0,) * len(shape))

    out = pl.pallas_call(
        functools.partial(_fused_gnn_kernel, nf=nf, d_out=d_out),
        grid=(bs, _NC // _NQ),
        in_specs=[
            pl.BlockSpec((1, rc, N * J),
                         (lambda k: (lambda b, c: (b, c * _NQ + k, 0)))(k))
            for k in range(_NQ)
        ] + [
            pl.BlockSpec((bs, N, x.shape[-1]), lambda b, c: (0, 0, 0)),
            pl.BlockSpec((bs, N, 1), lambda b, c: (0, 0, 0)),
            full(zt0), full(b0), full(zt1), full(b1), full(zt2), full(b2),
            full(ztl), full(bl),
        ],
        out_specs=pl.BlockSpec((1, N, d_out), lambda b, c: (b, 0, 0)),
        out_shape=jax.ShapeDtypeStruct((bs, N, d_out), jnp.float32),
        scratch_shapes=[
            pltpu.VMEM((N, N * J), jnp.bfloat16),
            pltpu.VMEM((N * J, _LANES), jnp.bfloat16),
            pltpu.VMEM((N, _LANES), jnp.float32),
        ],
        compiler_params=pltpu.CompilerParams(
            dimension_semantics=("arbitrary", "arbitrary"),
            vmem_limit_bytes=128 * 1024 * 1024,
        ),
    )(Wr, Wr, Wr, Wr, x, mask, zt0, b0, zt1, b1, zt2, b2, ztl, bl)
    return out


# speedup vs baseline: 1.1074x; 1.0731x over previous
import functools

import jax
import jax.numpy as jnp
from jax.experimental import pallas as pl
from jax.experimental.pallas import tpu as pltpu


_LANES = 128
_NC = 16
_NQ = 4


def _fused_gnn_kernel(W0_ref, W1_ref, W2_ref, W3_ref, x_ref, mask_ref,
                      zt0_ref, b0_ref, zt1_ref, b1_ref, zt2_ref, b2_ref,
                      ztl_ref, bl_ref, out_ref, wbf_ref, z0_ref, u0_ref,
                      nf: int, d_out: int):
    b_id = pl.program_id(0)
    c = pl.program_id(1)
    n = wbf_ref.shape[0]
    nj = wbf_ref.shape[1]
    rc = n // _NC

    def make_z(cur, zt):
        zwide = jnp.dot(cur, zt, preferred_element_type=jnp.float32)
        return zwide.reshape(nj, _LANES).astype(jnp.bfloat16)

    def act(u, b_row, mask):
        v = u + b_row
        lane = jax.lax.broadcasted_iota(jnp.int32, v.shape, 1)
        v = jnp.where(lane < nf, jnp.maximum(v, 0.0), v)
        return v * mask

    def big_dot_to_u0(z):
        for rb in range(_NC):
            u0_ref[pl.ds(rb * rc, rc)] = jnp.dot(
                wbf_ref[pl.ds(rb * rc, rc)], z,
                preferred_element_type=jnp.float32)

    @pl.when(c == 0)
    def _():
        z0_ref[...] = make_z(x_ref[b_id], zt0_ref[...])

    for k, wq in enumerate((W0_ref, W1_ref, W2_ref, W3_ref)):
        wblk = wq[0]
        row0 = (c * _NQ + k) * rc
        wbf_ref[pl.ds(row0, rc)] = wblk
        u0_ref[pl.ds(row0, rc)] = jnp.dot(wblk, z0_ref[...],
                                          preferred_element_type=jnp.float32)

    @pl.when(c == _NC // _NQ - 1)
    def _():
        mask = mask_ref[b_id]
        cur = act(u0_ref[...], b0_ref[...], mask)
        for zt_ref, br_ref in ((zt1_ref, b1_ref), (zt2_ref, b2_ref)):
            z = make_z(cur, zt_ref[...])
            big_dot_to_u0(z)
            cur = act(u0_ref[...], br_ref[...], mask)
        zl = make_z(cur, ztl_ref[...])
        big_dot_to_u0(zl)
        out_ref[0] = ((u0_ref[...] + bl_ref[...]) * mask)[:, :d_out]


def _zproj(w1, w2, j, pad_rows):
    w = jnp.concatenate([w1, w2], axis=0)
    return _zpad(w, j, pad_rows)


def _zpad(w, j, pad_rows):
    k = w.shape[0]
    d = w.shape[1] // j
    zt = w.reshape(k, j, d).transpose(2, 1, 0)
    zt = jnp.pad(zt, ((0, _LANES - d if pad_rows else 0),
                      (0, 0), (0, _LANES - k)))
    return zt.reshape(zt.shape[0], j * _LANES)


def kernel(W, x, mask, N_batch, fc1_w0, fc1_b0, fc2_w0, fc2_b0,
           fc1_w1, fc1_b1, fc2_w1, fc2_b1, fc1_w2, fc1_b2, fc2_w2, fc2_b2,
           fcl_w, fcl_b):
    bs, N, _, J = W.shape
    nf = fc1_b0.shape[0]
    d_out = fcl_w.shape[0]
    rc = N // _NC

    Wr = W.reshape(bs, N, N * J).astype(jnp.bfloat16)

    zt0 = _zproj(fc1_w0, fc2_w0, J, pad_rows=False)
    zt1 = _zproj(fc1_w1, fc2_w1, J, pad_rows=True)
    zt2 = _zproj(fc1_w2, fc2_w2, J, pad_rows=True)

    def bpad(b1_, b2_):
        b = jnp.concatenate([b1_, b2_])
        return jnp.pad(b, (0, _LANES - b.shape[0])).reshape(1, _LANES)

    b0 = bpad(fc1_b0, fc2_b0)
    b1 = bpad(fc1_b1, fc2_b1)
    b2 = bpad(fc1_b2, fc2_b2)
    ztl = _zpad(fcl_w, J, pad_rows=True)
    bl = jnp.pad(fcl_b, (0, _LANES - d_out)).reshape(1, _LANES)

    def full(a):
        shape = a.shape
        return pl.BlockSpec(shape, lambda b, c: (0,) * len(shape))

    out = pl.pallas_call(
        functools.partial(_fused_gnn_kernel, nf=nf, d_out=d_out),
        grid=(bs, _NC // _NQ),
        in_specs=[
            pl.BlockSpec((1, rc, N * J),
                         (lambda k: (lambda b, c: (b, c * _NQ + k, 0)))(k))
            for k in range(_NQ)
        ] + [
            pl.BlockSpec((bs, N, x.shape[-1]), lambda b, c: (0, 0, 0)),
            pl.BlockSpec((bs, N, 1), lambda b, c: (0, 0, 0)),
            full(zt0), full(b0), full(zt1), full(b1), full(zt2), full(b2),
            full(ztl), full(bl),
        ],
        out_specs=pl.BlockSpec((1, N, d_out), lambda b, c: (b, 0, 0)),
        out_shape=jax.ShapeDtypeStruct((bs, N, d_out), jnp.float32),
        scratch_shapes=[
            pltpu.VMEM((N, N * J), jnp.bfloat16),
            pltpu.VMEM((N * J, _LANES), jnp.bfloat16),
            pltpu.VMEM((N, _LANES), jnp.float32),
        ],
        compiler_params=pltpu.CompilerParams(
            dimension_semantics=(pltpu.ARBITRARY, pltpu.ARBITRARY),
            vmem_limit_bytes=128 * 1024 * 1024,
        ),
    )(Wr, Wr, Wr, Wr, x, mask, zt0, b0, zt1, b1, zt2, b2, ztl, bl)
    return out
